# baseline (speedup 1.0000x reference)
import functools

import jax
import jax.numpy as jnp
from jax import lax
from jax.experimental import pallas as pl
from jax.experimental.pallas import tpu as pltpu
from jax.experimental.pallas import tpu_sc as plsc

NUM_ENTITIES = 1000000
NUM_RELATIONS = 1000
EMBED_DIM = 64
BATCH = 16384
_HALF = NUM_ENTITIES // 2

_NC = 2
_NS = 16
_NW = _NC * _NS
_BPW = BATCH // _NW
_L = 16
_CH = 16
_NCH = _BPW // _CH


def _tec_body(heads_hbm, relations_hbm, tails_hbm, entity_hbm, relation_hbm,
              out_hbm, h_idx, r_idx, t_idx, h_bank, h_rem, t_bank, t_rem,
              rel_v, h_rows, t_rows, out_v, sem0, sem1, sem2, sem3):
    wid = lax.axis_index("s") * _NC + lax.axis_index("c")
    base = wid * _BPW

    pltpu.sync_copy(heads_hbm.at[pl.ds(base, _BPW)], h_idx)
    pltpu.sync_copy(relations_hbm.at[pl.ds(base, _BPW)], r_idx)
    pltpu.sync_copy(tails_hbm.at[pl.ds(base, _BPW)], t_idx)
    pltpu.sync_copy(relation_hbm, rel_v)

    sems = (sem0, sem1, sem2, sem3)

    def _split(k, carry):
        sl = pl.ds(k * _L, _L)
        hv = h_idx[sl]
        hb = jnp.where(hv >= _HALF, 1, 0)
        h_bank[sl] = hb
        h_rem[sl] = hv - hb * _HALF
        tv = t_idx[sl]
        tb = jnp.where(tv >= _HALF, 1, 0)
        t_bank[sl] = tb
        t_rem[sl] = tv - tb * _HALF
        return carry

    lax.fori_loop(0, _BPW // _L, _split, 0)

    def _fire(c, buf):
        hb = h_bank[pl.ds(c * _CH, _CH)]
        hr = h_rem[pl.ds(c * _CH, _CH)]
        tb = t_bank[pl.ds(c * _CH, _CH)]
        tr = t_rem[pl.ds(c * _CH, _CH)]
        for j in range(_CH):
            dst = buf * _CH + j
            pltpu.make_async_copy(
                entity_hbm.at[hb[j], pl.ds(hr[j], 1), :],
                h_rows.at[pl.ds(dst, 1), :],
                sems[buf]).start()
            pltpu.make_async_copy(
                entity_hbm.at[tb[j], pl.ds(tr[j], 1), :],
                t_rows.at[pl.ds(dst, 1), :],
                sems[buf]).start()

    def _drain(c, buf):
        half = pl.ds(buf * _CH, _CH)
        pltpu.make_async_copy(
            entity_hbm.at[0, pl.ds(0, _CH), :], h_rows.at[half, :],
            sems[buf]).wait()
        pltpu.make_async_copy(
            entity_hbm.at[0, pl.ds(0, _CH), :], t_rows.at[half, :],
            sems[buf]).wait()

    lanes = lax.iota(jnp.int32, _L)

    def _sqrt16(x):
        i = plsc.bitcast(x, jnp.int32)
        i = jnp.int32(0x5F3759DF) - lax.shift_right_logical(i, 1)
        y = plsc.bitcast(i, jnp.float32)
        xh = x * jnp.float32(0.5)
        for _ in range(3):
            y = y * (jnp.float32(1.5) - xh * y * y)
        return jnp.where(x > 0, x * y, jnp.float32(0.0))

    def _compute(c, buf):
        rows16 = jnp.int32(buf * _CH) + lanes
        rvec16 = r_idx[pl.ds(c * _CH, _CH)]
        rrow16 = lax.shift_right_logical(rvec16, 1)
        roff16 = lax.shift_left(jnp.bitwise_and(rvec16, 1), 6)

        def dim_step(d, acc):
            col = jnp.full((_L,), d, jnp.int32)
            hv = plsc.load_gather(h_rows, [rows16, col])
            rv = plsc.load_gather(rel_v, [rrow16, roff16 + d])
            tv = plsc.load_gather(t_rows, [rows16, col])
            dv = (hv + rv) - tv
            return acc + dv * dv

        acc = lax.fori_loop(0, EMBED_DIM, dim_step,
                            jnp.zeros((_L,), jnp.float32))
        out_v[pl.ds(c * _CH, _CH)] = _sqrt16(acc)

    _fire(0, 0)

    def step(cc, carry):
        c0 = cc * 2
        c1 = c0 + 1
        _fire(c1, 1)
        _drain(c0, 0)
        _compute(c0, 0)

        @pl.when(c1 + 1 < _NCH)
        def _():
            _fire(c1 + 1, 0)

        _drain(c1, 1)
        _compute(c1, 1)
        return carry

    lax.fori_loop(0, _NCH // 2, step, 0)

    pltpu.sync_copy(out_v, out_hbm.at[pl.ds(base, _BPW)])


@jax.jit
def _transe_sc(heads, relations, tails, entity_table, relation_table):
    mesh = plsc.VectorSubcoreMesh(core_axis_name="c", subcore_axis_name="s")
    kfn = functools.partial(
        pl.kernel,
        mesh=mesh,
        compiler_params=pltpu.CompilerParams(
            needs_layout_passes=False, use_tc_tiling_on_sc=True),
        out_type=jax.ShapeDtypeStruct((BATCH,), jnp.float32),
        scratch_types=[
            pltpu.VMEM((_BPW,), jnp.int32),
            pltpu.VMEM((_BPW,), jnp.int32),
            pltpu.VMEM((_BPW,), jnp.int32),
            pltpu.VMEM((_BPW,), jnp.int32),
            pltpu.VMEM((_BPW,), jnp.int32),
            pltpu.VMEM((_BPW,), jnp.int32),
            pltpu.VMEM((_BPW,), jnp.int32),
            pltpu.VMEM((NUM_RELATIONS // 2, 2 * EMBED_DIM), jnp.float32),
            pltpu.VMEM((4 * _CH, EMBED_DIM), jnp.float32),
            pltpu.VMEM((4 * _CH, EMBED_DIM), jnp.float32),
            pltpu.VMEM((_BPW,), jnp.float32),
            pltpu.SemaphoreType.DMA,
            pltpu.SemaphoreType.DMA,
            pltpu.SemaphoreType.DMA,
            pltpu.SemaphoreType.DMA,
        ],
    )(_tec_body)
    return kfn(heads, relations, tails, entity_table, relation_table)


def kernel(heads, relations, tails, entity_table, relation_table):
    heads = jnp.asarray(heads, jnp.int32)
    relations = jnp.asarray(relations, jnp.int32)
    tails = jnp.asarray(tails, jnp.int32)
    ent3 = entity_table.reshape(2, _HALF, EMBED_DIM)
    rel_p = relation_table.reshape(NUM_RELATIONS // 2, 2 * EMBED_DIM)
    return _transe_sc(heads, relations, tails, ent3, rel_p)


# speedup vs baseline: 1.0214x; 1.0214x over previous
import functools

import jax
import jax.numpy as jnp
from jax import lax
from jax.experimental import pallas as pl
from jax.experimental.pallas import tpu as pltpu
from jax.experimental.pallas import tpu_sc as plsc

NUM_ENTITIES = 1000000
NUM_RELATIONS = 1000
EMBED_DIM = 64
BATCH = 16384
_HALF = NUM_ENTITIES // 2

_NC = 2
_NS = 16
_NW = _NC * _NS
_BPW = BATCH // _NW
_L = 16
_CH = 16
_NCH = _BPW // _CH


def _tec_body(heads_hbm, relations_hbm, tails_hbm, entity_hbm, relation_hbm,
              out_hbm, h_idx, r_idx, t_idx, h_bank, h_rem, t_bank, t_rem,
              rel_v, h_rows, t_rows, out_v, sem0, sem1, sem2, sem3):
    wid = lax.axis_index("s") * _NC + lax.axis_index("c")
    base = wid * _BPW

    pltpu.sync_copy(heads_hbm.at[pl.ds(base, _BPW)], h_idx)
    pltpu.sync_copy(relations_hbm.at[pl.ds(base, _BPW)], r_idx)
    pltpu.sync_copy(tails_hbm.at[pl.ds(base, _BPW)], t_idx)
    pltpu.sync_copy(relation_hbm, rel_v)

    sems = (sem0, sem1, sem2, sem3)

    def _split(k, carry):
        sl = pl.ds(k * _L, _L)
        hv = h_idx[sl]
        hb = jnp.where(hv >= _HALF, 1, 0)
        h_bank[sl] = hb
        h_rem[sl] = hv - hb * _HALF
        tv = t_idx[sl]
        tb = jnp.where(tv >= _HALF, 1, 0)
        t_bank[sl] = tb
        t_rem[sl] = tv - tb * _HALF
        return carry

    lax.fori_loop(0, _BPW // _L, _split, 0)

    def _fire(c, buf):
        hb = h_bank[pl.ds(c * _CH, _CH)]
        hr = h_rem[pl.ds(c * _CH, _CH)]
        tb = t_bank[pl.ds(c * _CH, _CH)]
        tr = t_rem[pl.ds(c * _CH, _CH)]
        for j in range(_CH):
            dst = buf * _CH + j
            pltpu.make_async_copy(
                entity_hbm.at[hb[j], pl.ds(hr[j], 1), :],
                h_rows.at[pl.ds(dst, 1), :],
                sems[buf]).start()
            pltpu.make_async_copy(
                entity_hbm.at[tb[j], pl.ds(tr[j], 1), :],
                t_rows.at[pl.ds(dst, 1), :],
                sems[buf]).start()

    def _drain(c, buf):
        half = pl.ds(buf * _CH, _CH)
        pltpu.make_async_copy(
            entity_hbm.at[0, pl.ds(0, _CH), :], h_rows.at[half, :],
            sems[buf]).wait()
        pltpu.make_async_copy(
            entity_hbm.at[0, pl.ds(0, _CH), :], t_rows.at[half, :],
            sems[buf]).wait()

    lanes = lax.iota(jnp.int32, _L)

    def _sqrt16(x):
        i = plsc.bitcast(x, jnp.int32)
        i = jnp.int32(0x5F3759DF) - lax.shift_right_logical(i, 1)
        y = plsc.bitcast(i, jnp.float32)
        xh = x * jnp.float32(0.5)
        for _ in range(3):
            y = y * (jnp.float32(1.5) - xh * y * y)
        return jnp.where(x > 0, x * y, jnp.float32(0.0))

    def _compute(c, buf):
        rows16 = jnp.int32(buf * _CH) + lanes
        rvec16 = r_idx[pl.ds(c * _CH, _CH)]
        rrow16 = lax.shift_right_logical(rvec16, 1)
        roff16 = lax.shift_left(jnp.bitwise_and(rvec16, 1), 6)
        acc = jnp.zeros((_L,), jnp.float32)
        for d in range(EMBED_DIM):
            col = jnp.full((_L,), d, jnp.int32)
            hv = plsc.load_gather(h_rows, [rows16, col])
            rv = plsc.load_gather(rel_v, [rrow16, roff16 + d])
            tv = plsc.load_gather(t_rows, [rows16, col])
            dv = (hv + rv) - tv
            acc = acc + dv * dv
        out_v[pl.ds(c * _CH, _CH)] = _sqrt16(acc)

    _fire(0, 0)

    def step(cc, carry):
        c0 = cc * 2
        c1 = c0 + 1
        _fire(c1, 1)
        _drain(c0, 0)
        _compute(c0, 0)

        @pl.when(c1 + 1 < _NCH)
        def _():
            _fire(c1 + 1, 0)

        _drain(c1, 1)
        _compute(c1, 1)
        return carry

    lax.fori_loop(0, _NCH // 2, step, 0)

    pltpu.sync_copy(out_v, out_hbm.at[pl.ds(base, _BPW)])


@jax.jit
def _transe_sc(heads, relations, tails, entity_table, relation_table):
    mesh = plsc.VectorSubcoreMesh(core_axis_name="c", subcore_axis_name="s")
    kfn = functools.partial(
        pl.kernel,
        mesh=mesh,
        compiler_params=pltpu.CompilerParams(
            needs_layout_passes=False, use_tc_tiling_on_sc=True),
        out_type=jax.ShapeDtypeStruct((BATCH,), jnp.float32),
        scratch_types=[
            pltpu.VMEM((_BPW,), jnp.int32),
            pltpu.VMEM((_BPW,), jnp.int32),
            pltpu.VMEM((_BPW,), jnp.int32),
            pltpu.VMEM((_BPW,), jnp.int32),
            pltpu.VMEM((_BPW,), jnp.int32),
            pltpu.VMEM((_BPW,), jnp.int32),
            pltpu.VMEM((_BPW,), jnp.int32),
            pltpu.VMEM((NUM_RELATIONS // 2, 2 * EMBED_DIM), jnp.float32),
            pltpu.VMEM((4 * _CH, EMBED_DIM), jnp.float32),
            pltpu.VMEM((4 * _CH, EMBED_DIM), jnp.float32),
            pltpu.VMEM((_BPW,), jnp.float32),
            pltpu.SemaphoreType.DMA,
            pltpu.SemaphoreType.DMA,
            pltpu.SemaphoreType.DMA,
            pltpu.SemaphoreType.DMA,
        ],
    )(_tec_body)
    return kfn(heads, relations, tails, entity_table, relation_table)


def kernel(heads, relations, tails, entity_table, relation_table):
    heads = jnp.asarray(heads, jnp.int32)
    relations = jnp.asarray(relations, jnp.int32)
    tails = jnp.asarray(tails, jnp.int32)
    ent3 = entity_table.reshape(2, _HALF, EMBED_DIM)
    rel_p = relation_table.reshape(NUM_RELATIONS // 2, 2 * EMBED_DIM)
    return _transe_sc(heads, relations, tails, ent3, rel_p)


# speedup vs baseline: 1.0222x; 1.0008x over previous
import functools

import jax
import jax.numpy as jnp
from jax import lax
from jax.experimental import pallas as pl
from jax.experimental.pallas import tpu as pltpu
from jax.experimental.pallas import tpu_sc as plsc

NUM_ENTITIES = 1000000
NUM_RELATIONS = 1000
EMBED_DIM = 64
BATCH = 16384
_HALF = NUM_ENTITIES // 2

_NC = 2
_NS = 16
_NW = _NC * _NS
_BPW = BATCH // _NW
_L = 16
_CH = 16
_NCH = _BPW // _CH


def _tec_body(heads_hbm, relations_hbm, tails_hbm, entity_hbm, relation_hbm,
              out_hbm, h_idx, r_idx, t_idx, h_bank, h_rem, t_bank, t_rem,
              rel_v, h_rows, t_rows, out_v, sem0, sem1):
    wid = lax.axis_index("s") * _NC + lax.axis_index("c")
    base = wid * _BPW

    pltpu.sync_copy(heads_hbm.at[pl.ds(base, _BPW)], h_idx)
    pltpu.sync_copy(relations_hbm.at[pl.ds(base, _BPW)], r_idx)
    pltpu.sync_copy(tails_hbm.at[pl.ds(base, _BPW)], t_idx)
    pltpu.sync_copy(relation_hbm, rel_v)

    sems = (sem0, sem1)

    def _split(k, carry):
        sl = pl.ds(k * _L, _L)
        hv = h_idx[sl]
        hb = jnp.where(hv >= _HALF, 1, 0)
        h_bank[sl] = hb
        h_rem[sl] = hv - hb * _HALF
        tv = t_idx[sl]
        tb = jnp.where(tv >= _HALF, 1, 0)
        t_bank[sl] = tb
        t_rem[sl] = tv - tb * _HALF
        return carry

    lax.fori_loop(0, _BPW // _L, _split, 0)

    def _fire(c, buf):
        hb = h_bank[pl.ds(c * _CH, _CH)]
        hr = h_rem[pl.ds(c * _CH, _CH)]
        tb = t_bank[pl.ds(c * _CH, _CH)]
        tr = t_rem[pl.ds(c * _CH, _CH)]
        for j in range(_CH):
            dst = buf * _CH + j
            pltpu.make_async_copy(
                entity_hbm.at[hb[j], pl.ds(hr[j], 1), :],
                h_rows.at[pl.ds(dst, 1), :],
                sems[buf]).start()
            pltpu.make_async_copy(
                entity_hbm.at[tb[j], pl.ds(tr[j], 1), :],
                t_rows.at[pl.ds(dst, 1), :],
                sems[buf]).start()

    def _drain(c, buf):
        half = pl.ds(buf * _CH, _CH)
        pltpu.make_async_copy(
            entity_hbm.at[0, pl.ds(0, _CH), :], h_rows.at[half, :],
            sems[buf]).wait()
        pltpu.make_async_copy(
            entity_hbm.at[0, pl.ds(0, _CH), :], t_rows.at[half, :],
            sems[buf]).wait()

    lanes = lax.iota(jnp.int32, _L)

    def _sqrt16(x):
        i = plsc.bitcast(x, jnp.int32)
        i = jnp.int32(0x5F3759DF) - lax.shift_right_logical(i, 1)
        y = plsc.bitcast(i, jnp.float32)
        xh = x * jnp.float32(0.5)
        for _ in range(3):
            y = y * (jnp.float32(1.5) - xh * y * y)
        return jnp.where(x > 0, x * y, jnp.float32(0.0))

    def _compute(c, buf):
        rows16 = jnp.int32(buf * _CH) + lanes
        rvec16 = r_idx[pl.ds(c * _CH, _CH)]
        rrow16 = lax.shift_right_logical(rvec16, 1)
        roff16 = lax.shift_left(jnp.bitwise_and(rvec16, 1), 6)
        acc = jnp.zeros((_L,), jnp.float32)
        for d in range(EMBED_DIM):
            col = jnp.full((_L,), d, jnp.int32)
            hv = plsc.load_gather(h_rows, [rows16, col])
            rv = plsc.load_gather(rel_v, [rrow16, roff16 + d])
            tv = plsc.load_gather(t_rows, [rows16, col])
            dv = (hv + rv) - tv
            acc = acc + dv * dv
        out_v[pl.ds(c * _CH, _CH)] = _sqrt16(acc)

    _fire(0, 0)

    def step(cc, carry):
        c0 = cc * 2
        c1 = c0 + 1
        _fire(c1, 1)
        _drain(c0, 0)
        _compute(c0, 0)

        @pl.when(c1 + 1 < _NCH)
        def _():
            _fire(c1 + 1, 0)

        _drain(c1, 1)
        _compute(c1, 1)
        return carry

    lax.fori_loop(0, _NCH // 2, step, 0)

    pltpu.sync_copy(out_v, out_hbm.at[pl.ds(base, _BPW)])


@jax.jit
def _transe_sc(heads, relations, tails, entity_table, relation_table):
    mesh = plsc.VectorSubcoreMesh(core_axis_name="c", subcore_axis_name="s")
    kfn = functools.partial(
        pl.kernel,
        mesh=mesh,
        compiler_params=pltpu.CompilerParams(
            needs_layout_passes=False, use_tc_tiling_on_sc=True),
        out_type=jax.ShapeDtypeStruct((BATCH,), jnp.float32),
        scratch_types=[
            pltpu.VMEM((_BPW,), jnp.int32),
            pltpu.VMEM((_BPW,), jnp.int32),
            pltpu.VMEM((_BPW,), jnp.int32),
            pltpu.VMEM((_BPW,), jnp.int32),
            pltpu.VMEM((_BPW,), jnp.int32),
            pltpu.VMEM((_BPW,), jnp.int32),
            pltpu.VMEM((_BPW,), jnp.int32),
            pltpu.VMEM((NUM_RELATIONS // 2, 2 * EMBED_DIM), jnp.float32),
            pltpu.VMEM((2 * _CH, EMBED_DIM), jnp.float32),
            pltpu.VMEM((2 * _CH, EMBED_DIM), jnp.float32),
            pltpu.VMEM((_BPW,), jnp.float32),
            pltpu.SemaphoreType.DMA,
            pltpu.SemaphoreType.DMA,
        ],
    )(_tec_body)
    return kfn(heads, relations, tails, entity_table, relation_table)


def kernel(heads, relations, tails, entity_table, relation_table):
    heads = jnp.asarray(heads, jnp.int32)
    relations = jnp.asarray(relations, jnp.int32)
    tails = jnp.asarray(tails, jnp.int32)
    ent3 = entity_table.reshape(2, _HALF, EMBED_DIM)
    rel_p = relation_table.reshape(NUM_RELATIONS // 2, 2 * EMBED_DIM)
    return _transe_sc(heads, relations, tails, ent3, rel_p)
